# baseline (speedup 1.0000x reference)
import jax
import jax.numpy as jnp
from jax import lax
from jax.experimental import pallas as pl
from jax.experimental.pallas import tpu as pltpu
from jax.experimental.pallas import tpu_sc as plsc

_BATCH = 4096
_NUM_SAMPLES = 1024
_DIM = 64
_ROW = _DIM + 1
_TINY = 1e-13

_NUM_CORES = 2
_NUM_SUBCORES = 16
_NUM_WORKERS = _NUM_CORES * _NUM_SUBCORES
_LANES = 128
_NBUF = 4

_BLK = 1024


def _make_sc_gather(t_off, t_count, s_count):
    total = t_count + s_count
    ipw = total // _NUM_WORKERS
    assert ipw * _NUM_WORKERS == total and ipw % _NBUF == 0
    nchunk = ipw // _NBUF
    w_split = t_count // ipw
    t_split = t_count - w_split * ipw

    def body(tgt_hbm, smp_hbm, tableT_hbm, out_hbm, ids_v, stage_v,
             out_sem, *bufs_and_sems):
        blks = bufs_and_sems[:_NBUF]
        sems = bufs_and_sems[_NBUF:]
        wid = lax.axis_index("s") * _NUM_CORES + lax.axis_index("c")
        base = wid * ipw

        if s_count == 0:
            pltpu.sync_copy(tgt_hbm.at[pl.ds(t_off + base, ipw)],
                            ids_v.at[pl.ds(0, ipw)])
        else:
            @pl.when(wid < w_split)
            def _load_targets():
                pltpu.sync_copy(tgt_hbm.at[pl.ds(t_off + base, ipw)],
                                ids_v.at[pl.ds(0, ipw)])

            @pl.when(wid == w_split)
            def _load_straddle():
                pltpu.sync_copy(tgt_hbm.at[pl.ds(t_off + base, t_split)],
                                ids_v.at[pl.ds(0, t_split)])
                pltpu.sync_copy(smp_hbm.at[pl.ds(0, ipw - t_split)],
                                ids_v.at[pl.ds(t_split, ipw - t_split)])

            @pl.when(wid > w_split)
            def _load_samples():
                pltpu.sync_copy(smp_hbm.at[pl.ds(base - t_count, ipw)],
                                ids_v.at[pl.ds(0, ipw)])

        def issue(col, k):
            aligned = pl.multiple_of(col & -_LANES, _LANES)
            pltpu.async_copy(
                tableT_hbm.at[:, pl.ds(aligned, _LANES)],
                blks[k].at[pl.ds(0, _ROW)],
                sems[k],
            )

        idvec0 = ids_v[pl.ds(0, 16)]
        for k in range(_NBUF):
            issue(idvec0[k], k)

        @pl.loop(0, nchunk)
        def _step(ch):
            idvec_cur = ids_v[pl.ds(ch * _NBUF, 16)]
            idvec_nxt = ids_v[pl.ds((ch + 1) * _NBUF, 16)]
            for k in range(_NBUF):
                pltpu.make_async_copy(
                    tableT_hbm.at[:, pl.ds(0, _LANES)],
                    blks[k].at[pl.ds(0, _ROW)],
                    sems[k],
                ).wait()
                lane = jnp.broadcast_to(idvec_cur[k] & (_LANES - 1), (16,))
                for c in range(5):
                    rows = c * 16 + lax.iota(jnp.int32, 16)
                    vals = plsc.load_gather(blks[k], [rows, lane])
                    stage_v.at[ch * _NBUF + k, pl.ds(c * 16, 16)][...] = vals
                pltpu.async_copy(
                    stage_v.at[ch * _NBUF + k, pl.ds(0, _ROW)],
                    out_hbm.at[base + ch * _NBUF + k],
                    out_sem,
                )

                @pl.when(ch < nchunk - 1)
                def _next():
                    issue(idvec_nxt[k], k)

        @pl.loop(0, ipw)
        def _drain(j):
            pltpu.make_async_copy(
                stage_v.at[0, pl.ds(0, _ROW)], out_hbm.at[base],
                out_sem).wait()

    def run(targets, samples, tableT):
        return pl.kernel(
            body,
            out_type=jax.ShapeDtypeStruct((total, _ROW), jnp.float32),
            mesh=plsc.VectorSubcoreMesh(
                core_axis_name="c", subcore_axis_name="s"),
            scratch_types=(
                [
                    pltpu.VMEM((ipw + 16,), jnp.int32),
                    pltpu.VMEM((ipw, 80), jnp.float32),
                    pltpu.SemaphoreType.DMA,
                ]
                + [pltpu.VMEM((80, _LANES), jnp.float32)] * _NBUF
                + [pltpu.SemaphoreType.DMA] * _NBUF
            ),
            compiler_params=pltpu.CompilerParams(
                use_tc_tiling_on_sc=True, needs_layout_passes=False),
        )(targets, samples, tableT)

    return run


_sc_gather_all = _make_sc_gather(0, _BATCH, _NUM_SAMPLES)


def _loss_body(ext_ref, trow_ref, srow_ref, tgt_ref, smp_ref, tec_ref,
               sec_ref, out_ref):
    i = pl.program_id(0)
    ext = ext_ref[...]
    trow = trow_ref[...]
    srow = srow_ref[...]
    t_logit = (jnp.sum(ext * trow, axis=1, keepdims=True)
               - jnp.log(tec_ref[...] + _TINY))
    s_log = lax.dot_general(
        ext, srow, (((1,), (1,)), ((), ())),
        preferred_element_type=jnp.float32)
    s_log = s_log - jnp.log(sec_ref[...] + _TINY)
    s_log = jnp.where(tgt_ref[...] == smp_ref[...], -10000.0, s_log)
    m = jnp.maximum(jnp.max(s_log, axis=1, keepdims=True), t_logit)
    ssum = (jnp.sum(jnp.exp(s_log - m), axis=1, keepdims=True)
            + jnp.exp(t_logit - m))
    lse = m + jnp.log(ssum)
    partial = jnp.sum(lse - t_logit)

    @pl.when(i == 0)
    def _init():
        out_ref[0, 0] = 0.0

    out_ref[0, 0] += partial


def kernel(embeddings, targets, samples, target_expected_count,
           sampled_expected_count, table):
    rows = _sc_gather_all(targets, samples, table.T)

    ones = jnp.ones((_BATCH, 1), jnp.float32)
    ext = jnp.concatenate([ones, embeddings], axis=1)

    grid = _BATCH // _BLK
    out = pl.pallas_call(
        _loss_body,
        grid=(grid,),
        in_specs=[
            pl.BlockSpec((_BLK, _ROW), lambda i: (i, 0)),
            pl.BlockSpec((_BLK, _ROW), lambda i: (i, 0)),
            pl.BlockSpec((_NUM_SAMPLES, _ROW),
                         lambda i: (_BATCH // _NUM_SAMPLES, 0)),
            pl.BlockSpec((_BLK, 1), lambda i: (i, 0)),
            pl.BlockSpec((1, _NUM_SAMPLES), lambda i: (0, 0)),
            pl.BlockSpec((_BLK, 1), lambda i: (i, 0)),
            pl.BlockSpec((1, _NUM_SAMPLES), lambda i: (0, 0)),
        ],
        out_specs=pl.BlockSpec(memory_space=pltpu.SMEM),
        out_shape=jax.ShapeDtypeStruct((1, 1), jnp.float32),
    )(
        ext,
        rows,
        rows,
        targets.reshape(_BATCH, 1),
        samples.reshape(1, _NUM_SAMPLES),
        target_expected_count.reshape(_BATCH, 1),
        sampled_expected_count.reshape(1, _NUM_SAMPLES),
    )
    return out[0, 0]


# speedup vs baseline: 1.1202x; 1.1202x over previous
import jax
import jax.numpy as jnp
from jax import lax
from jax.experimental import pallas as pl
from jax.experimental.pallas import tpu as pltpu
from jax.experimental.pallas import tpu_sc as plsc

_BATCH = 4096
_NUM_SAMPLES = 1024
_DIM = 64
_ROW = _DIM + 1
_TINY = 1e-13

_NUM_CORES = 2
_NUM_SUBCORES = 16
_NUM_WORKERS = _NUM_CORES * _NUM_SUBCORES
_LANES = 128
_NBUF = 8

_BLK = 1024


def _make_sc_gather(t_off, t_count, s_count):
    total = t_count + s_count
    ipw = total // _NUM_WORKERS
    assert ipw * _NUM_WORKERS == total and ipw % _NBUF == 0
    nchunk = ipw // _NBUF
    w_split = t_count // ipw
    t_split = t_count - w_split * ipw

    def body(tgt_hbm, smp_hbm, tableT_hbm, out_hbm, ids_v, stage_v,
             out_sem, *bufs_and_sems):
        blks = bufs_and_sems[:_NBUF]
        sems = bufs_and_sems[_NBUF:]
        wid = lax.axis_index("s") * _NUM_CORES + lax.axis_index("c")
        base = wid * ipw

        if s_count == 0:
            pltpu.sync_copy(tgt_hbm.at[pl.ds(t_off + base, ipw)],
                            ids_v.at[pl.ds(0, ipw)])
        else:
            @pl.when(wid < w_split)
            def _load_targets():
                pltpu.sync_copy(tgt_hbm.at[pl.ds(t_off + base, ipw)],
                                ids_v.at[pl.ds(0, ipw)])

            @pl.when(wid == w_split)
            def _load_straddle():
                pltpu.sync_copy(tgt_hbm.at[pl.ds(t_off + base, t_split)],
                                ids_v.at[pl.ds(0, t_split)])
                pltpu.sync_copy(smp_hbm.at[pl.ds(0, ipw - t_split)],
                                ids_v.at[pl.ds(t_split, ipw - t_split)])

            @pl.when(wid > w_split)
            def _load_samples():
                pltpu.sync_copy(smp_hbm.at[pl.ds(base - t_count, ipw)],
                                ids_v.at[pl.ds(0, ipw)])

        def issue(col, k):
            aligned = pl.multiple_of(col & -_LANES, _LANES)
            pltpu.async_copy(
                tableT_hbm.at[:, pl.ds(aligned, _LANES)],
                blks[k].at[pl.ds(0, _ROW)],
                sems[k],
            )

        idvec0 = ids_v[pl.ds(0, 16)]
        for k in range(_NBUF):
            issue(idvec0[k], k)

        @pl.loop(0, nchunk)
        def _step(ch):
            idvec_cur = ids_v[pl.ds(ch * _NBUF, 16)]
            idvec_nxt = ids_v[pl.ds((ch + 1) * _NBUF, 16)]
            for k in range(_NBUF):
                pltpu.make_async_copy(
                    tableT_hbm.at[:, pl.ds(0, _LANES)],
                    blks[k].at[pl.ds(0, _ROW)],
                    sems[k],
                ).wait()
                lane = jnp.broadcast_to(idvec_cur[k] & (_LANES - 1), (16,))
                for c in range(5):
                    rows = c * 16 + lax.iota(jnp.int32, 16)
                    vals = plsc.load_gather(blks[k], [rows, lane])
                    stage_v.at[ch * _NBUF + k, pl.ds(c * 16, 16)][...] = vals
                pltpu.async_copy(
                    stage_v.at[ch * _NBUF + k, pl.ds(0, _ROW)],
                    out_hbm.at[base + ch * _NBUF + k],
                    out_sem,
                )

                @pl.when(ch < nchunk - 1)
                def _next():
                    issue(idvec_nxt[k], k)

        @pl.loop(0, ipw)
        def _drain(j):
            pltpu.make_async_copy(
                stage_v.at[0, pl.ds(0, _ROW)], out_hbm.at[base],
                out_sem).wait()

    def run(targets, samples, tableT):
        return pl.kernel(
            body,
            out_type=jax.ShapeDtypeStruct((total, _ROW), jnp.float32),
            mesh=plsc.VectorSubcoreMesh(
                core_axis_name="c", subcore_axis_name="s"),
            scratch_types=(
                [
                    pltpu.VMEM((ipw + 16,), jnp.int32),
                    pltpu.VMEM((ipw, 80), jnp.float32),
                    pltpu.SemaphoreType.DMA,
                ]
                + [pltpu.VMEM((80, _LANES), jnp.float32)] * _NBUF
                + [pltpu.SemaphoreType.DMA] * _NBUF
            ),
            compiler_params=pltpu.CompilerParams(
                use_tc_tiling_on_sc=True, needs_layout_passes=False),
        )(targets, samples, tableT)

    return run


_sc_gather_all = _make_sc_gather(0, _BATCH, _NUM_SAMPLES)


def _loss_body(ext_ref, trow_ref, srow_ref, tgt_ref, smp_ref, tec_ref,
               sec_ref, out_ref):
    i = pl.program_id(0)
    ext = ext_ref[...]
    trow = trow_ref[...]
    srow = srow_ref[...]
    t_logit = (jnp.sum(ext * trow, axis=1, keepdims=True)
               - jnp.log(tec_ref[...] + _TINY))
    s_log = lax.dot_general(
        ext.astype(jnp.bfloat16), srow.astype(jnp.bfloat16),
        (((1,), (1,)), ((), ())),
        preferred_element_type=jnp.float32)
    s_log = s_log - jnp.log(sec_ref[...] + _TINY)
    s_log = jnp.where(tgt_ref[...] == smp_ref[...], -10000.0, s_log)
    m = jnp.maximum(jnp.max(s_log, axis=1, keepdims=True), t_logit)
    ssum = (jnp.sum(jnp.exp(s_log - m), axis=1, keepdims=True)
            + jnp.exp(t_logit - m))
    lse = m + jnp.log(ssum)
    partial = jnp.sum(lse - t_logit)

    @pl.when(i == 0)
    def _init():
        out_ref[0, 0] = 0.0

    out_ref[0, 0] += partial


def kernel(embeddings, targets, samples, target_expected_count,
           sampled_expected_count, table):
    rows = _sc_gather_all(targets, samples, table.T)

    ones = jnp.ones((_BATCH, 1), jnp.float32)
    ext = jnp.concatenate([ones, embeddings], axis=1)

    grid = _BATCH // _BLK
    out = pl.pallas_call(
        _loss_body,
        grid=(grid,),
        in_specs=[
            pl.BlockSpec((_BLK, _ROW), lambda i: (i, 0)),
            pl.BlockSpec((_BLK, _ROW), lambda i: (i, 0)),
            pl.BlockSpec((_NUM_SAMPLES, _ROW),
                         lambda i: (_BATCH // _NUM_SAMPLES, 0)),
            pl.BlockSpec((_BLK, 1), lambda i: (i, 0)),
            pl.BlockSpec((1, _NUM_SAMPLES), lambda i: (0, 0)),
            pl.BlockSpec((_BLK, 1), lambda i: (i, 0)),
            pl.BlockSpec((1, _NUM_SAMPLES), lambda i: (0, 0)),
        ],
        out_specs=pl.BlockSpec(memory_space=pltpu.SMEM),
        out_shape=jax.ShapeDtypeStruct((1, 1), jnp.float32),
    )(
        ext,
        rows,
        rows,
        targets.reshape(_BATCH, 1),
        samples.reshape(1, _NUM_SAMPLES),
        target_expected_count.reshape(_BATCH, 1),
        sampled_expected_count.reshape(1, _NUM_SAMPLES),
    )
    return out[0, 0]
